# baseline (speedup 1.0000x reference)
import functools

import jax
import jax.numpy as jnp
from jax import lax
from jax.experimental import pallas as pl
from jax.experimental.pallas import tpu as pltpu
from jax.experimental.pallas import tpu_sc as plsc

_V = 1000000
_D = 32
_B = 16384
_L = 16
_CH = 128
_W = 4 * _D


@functools.lru_cache(maxsize=1)
def _build_sc_dots():
    info = plsc.get_sparse_core_info()
    NC, NS = info.num_cores, info.num_subcores
    NW = NC * NS
    bpw = _B // NW
    nch = bpw // _CH
    groups = _CH // _L

    mesh = plsc.VectorSubcoreMesh(core_axis_name="c", subcore_axis_name="s")

    @functools.partial(
        pl.kernel,
        out_type=[
            jax.ShapeDtypeStruct((NW, nch, _CH), jnp.float32),
            jax.ShapeDtypeStruct((NW, nch, _CH), jnp.float32),
        ],
        mesh=mesh,
        scratch_types=[
            pltpu.VMEM((nch, _CH), jnp.int32),
            pltpu.VMEM((nch, _CH), jnp.int32),
            pltpu.VMEM((nch, _CH), jnp.int32),
            pltpu.VMEM((nch, _CH), jnp.int32),
            pltpu.VMEM((nch, _CH), jnp.int32),
            pltpu.VMEM((nch, _CH), jnp.int32),
            pltpu.VMEM((_CH, _W), jnp.float32),
            pltpu.VMEM((_CH, _W), jnp.float32),
            pltpu.VMEM((_CH, _W), jnp.float32),
            pltpu.VMEM((nch, _CH), jnp.float32),
            pltpu.VMEM((nch, _CH), jnp.float32),
            pltpu.SemaphoreType.DMA,
        ],
        compiler_params=pltpu.CompilerParams(needs_layout_passes=False),
    )
    def dots(iw_hbm, ow_hbm, nw_hbm, iw4_hbm, ow4_hbm, nw4_hbm,
             ei_hbm, eo_hbm, do_hbm, dn_hbm,
             iw_v, ow_v, nw_v, iw4_v, ow4_v, nw4_v,
             iv_v, ov_v, nv_v, do_v, dn_v, sem):
        wid = lax.axis_index("s") * NC + lax.axis_index("c")
        pltpu.sync_copy(iw_hbm.at[wid], iw_v)
        pltpu.sync_copy(ow_hbm.at[wid], ow_v)
        pltpu.sync_copy(nw_hbm.at[wid], nw_v)
        pltpu.sync_copy(iw4_hbm.at[wid], iw4_v)
        pltpu.sync_copy(ow4_hbm.at[wid], ow4_v)
        pltpu.sync_copy(nw4_hbm.at[wid], nw4_v)

        for c in range(nch):
            cps = [
                pltpu.async_copy(ei_hbm.at[iw4_v.at[c]], iv_v, sem),
                pltpu.async_copy(eo_hbm.at[ow4_v.at[c]], ov_v, sem),
                pltpu.async_copy(eo_hbm.at[nw4_v.at[c]], nv_v, sem),
            ]
            for cp in cps:
                cp.wait()

            for g in range(groups):
                sl = pl.ds(g * _L, _L)
                iwv = iw_v[c, sl]
                owv = ow_v[c, sl]
                nwv = nw_v[c, sl]
                rows = g * _L + lax.iota(jnp.int32, _L)
                icol = (iwv & 3) * _D
                ocol = (owv & 3) * _D
                ncol = (nwv & 3) * _D
                acc_o = jnp.zeros((_L,), jnp.float32)
                acc_n = jnp.zeros((_L,), jnp.float32)
                for d in range(_D):
                    iv = plsc.load_gather(iv_v, [rows, icol + d])
                    ov = plsc.load_gather(ov_v, [rows, ocol + d])
                    nv = plsc.load_gather(nv_v, [rows, ncol + d])
                    acc_o = acc_o + iv * ov
                    acc_n = acc_n + iv * nv
                do_v[c, sl] = acc_o
                dn_v[c, sl] = acc_n

        pltpu.sync_copy(do_v, do_hbm.at[wid])
        pltpu.sync_copy(dn_v, dn_hbm.at[wid])

    return dots, NW, nch


def _loss_body(do_ref, dn_ref, out_ref):
    x = do_ref[...]
    y = -dn_ref[...]
    ls = jnp.minimum(x, 0.0) - jnp.log1p(jnp.exp(-jnp.abs(x)))
    ls = ls + jnp.minimum(y, 0.0) - jnp.log1p(jnp.exp(-jnp.abs(y)))
    out_ref[0, 0] = -jnp.sum(ls) / _B


@functools.lru_cache(maxsize=1)
def _build_loss():
    return pl.pallas_call(
        _loss_body,
        out_shape=jax.ShapeDtypeStruct((1, 1), jnp.float32),
        out_specs=pl.BlockSpec(memory_space=pltpu.SMEM),
    )


@jax.jit
def kernel(iword, owords, nwords, emb_i, emb_o):
    dots, NW, nch = _build_sc_dots()
    iw = iword.astype(jnp.int32).reshape(NW, nch, _CH)
    ow = owords.astype(jnp.int32).reshape(NW, nch, _CH)
    nw = nwords.astype(jnp.int32).reshape(NW, nch, _CH)
    do, dn = dots(iw, ow, nw, iw >> 2, ow >> 2, nw >> 2,
                  emb_i.reshape(_V // 4, _W), emb_o.reshape(_V // 4, _W))
    loss = _build_loss()(do.reshape(128, 128), dn.reshape(128, 128))
    return loss[0, 0]


# speedup vs baseline: 1.0158x; 1.0158x over previous
import functools

import jax
import jax.numpy as jnp
from jax import lax
from jax.experimental import pallas as pl
from jax.experimental.pallas import tpu as pltpu
from jax.experimental.pallas import tpu_sc as plsc

_V = 1000000
_D = 32
_B = 16384
_L = 16
_CH = 128


@functools.lru_cache(maxsize=1)
def _build_sc_dots():
    info = plsc.get_sparse_core_info()
    NC, NS = info.num_cores, info.num_subcores
    NW = NC * NS
    bpw = _B // NW
    nch = bpw // _CH
    groups = bpw // _L

    mesh = plsc.VectorSubcoreMesh(core_axis_name="c", subcore_axis_name="s")

    @functools.partial(
        pl.kernel,
        out_type=[
            jax.ShapeDtypeStruct((_B,), jnp.float32),
            jax.ShapeDtypeStruct((_B,), jnp.float32),
        ],
        mesh=mesh,
        scratch_types=[
            pltpu.VMEM((nch, _CH), jnp.int32),
            pltpu.VMEM((nch, _CH), jnp.int32),
            pltpu.VMEM((nch, _CH), jnp.int32),
            pltpu.VMEM((bpw, _D), jnp.float32),
            pltpu.VMEM((bpw, _D), jnp.float32),
            pltpu.VMEM((bpw, _D), jnp.float32),
            pltpu.VMEM((bpw,), jnp.float32),
            pltpu.VMEM((bpw,), jnp.float32),
            pltpu.SemaphoreType.DMA,
        ],
        compiler_params=pltpu.CompilerParams(
            use_tc_tiling_on_sc=False, needs_layout_passes=False
        ),
    )
    def dots(iw_hbm, ow_hbm, nw_hbm, ei_hbm, eo_hbm, do_hbm, dn_hbm,
             iw_v, ow_v, nw_v, iv_v, ov_v, nv_v, do_v, dn_v, sem):
        wid = lax.axis_index("s") * NC + lax.axis_index("c")
        pltpu.sync_copy(iw_hbm.at[wid], iw_v)
        pltpu.sync_copy(ow_hbm.at[wid], ow_v)
        pltpu.sync_copy(nw_hbm.at[wid], nw_v)
        cps = []
        for c in range(nch):
            dst = pl.ds(c * _CH, _CH)
            cps.append(pltpu.async_copy(ei_hbm.at[iw_v.at[c]], iv_v.at[dst], sem))
            cps.append(pltpu.async_copy(eo_hbm.at[ow_v.at[c]], ov_v.at[dst], sem))
            cps.append(pltpu.async_copy(eo_hbm.at[nw_v.at[c]], nv_v.at[dst], sem))
        for cp in cps:
            cp.wait()

        def group_body(g, carry):
            rows = g * _L + lax.iota(jnp.int32, _L)
            acc_o = jnp.zeros((_L,), jnp.float32)
            acc_n = jnp.zeros((_L,), jnp.float32)
            for d in range(_D):
                dd = jnp.full((_L,), d, jnp.int32)
                iv = plsc.load_gather(iv_v, [rows, dd])
                ov = plsc.load_gather(ov_v, [rows, dd])
                nv = plsc.load_gather(nv_v, [rows, dd])
                acc_o = acc_o + iv * ov
                acc_n = acc_n + iv * nv
            do_v[pl.ds(g * _L, _L)] = acc_o
            dn_v[pl.ds(g * _L, _L)] = acc_n
            return carry

        lax.fori_loop(0, groups, group_body, 0)
        pltpu.sync_copy(do_v, do_hbm.at[pl.ds(wid * bpw, bpw)])
        pltpu.sync_copy(dn_v, dn_hbm.at[pl.ds(wid * bpw, bpw)])

    return dots, NW, nch


def _loss_body(do_ref, dn_ref, out_ref):
    x = do_ref[...]
    y = -dn_ref[...]
    ls = jnp.minimum(x, 0.0) - jnp.log1p(jnp.exp(-jnp.abs(x)))
    ls = ls + jnp.minimum(y, 0.0) - jnp.log1p(jnp.exp(-jnp.abs(y)))
    out_ref[0, 0] = -jnp.sum(ls) / _B


@functools.lru_cache(maxsize=1)
def _build_loss():
    return pl.pallas_call(
        _loss_body,
        out_shape=jax.ShapeDtypeStruct((1, 1), jnp.float32),
        out_specs=pl.BlockSpec(memory_space=pltpu.SMEM),
    )


@jax.jit
def kernel(iword, owords, nwords, emb_i, emb_o):
    dots, NW, nch = _build_sc_dots()
    iw = iword.astype(jnp.int32).reshape(NW, nch, _CH)
    ow = owords.astype(jnp.int32).reshape(NW, nch, _CH)
    nw = nwords.astype(jnp.int32).reshape(NW, nch, _CH)
    do, dn = dots(iw, ow, nw, emb_i, emb_o)
    loss = _build_loss()(do.reshape(128, 128), dn.reshape(128, 128))
    return loss[0, 0]


# speedup vs baseline: 3.1913x; 3.1417x over previous
import functools

import jax
import jax.numpy as jnp
from jax import lax
from jax.experimental import pallas as pl
from jax.experimental.pallas import tpu as pltpu
from jax.experimental.pallas import tpu_sc as plsc

_V = 1000000
_D = 32
_B = 16384
_L = 16
_RING = 8


def _splat(x):
    return jnp.full((_L,), x, jnp.int32)


@functools.lru_cache(maxsize=1)
def _build_sc_dots():
    info = plsc.get_sparse_core_info()
    NC, NS = info.num_cores, info.num_subcores
    NW = NC * NS
    bpw = _B // NW
    nrow = bpw // 128

    mesh = plsc.VectorSubcoreMesh(core_axis_name="c", subcore_axis_name="s")

    @functools.partial(
        pl.kernel,
        out_type=[
            jax.ShapeDtypeStruct((NW, nrow, 128), jnp.float32),
            jax.ShapeDtypeStruct((NW, nrow, 128), jnp.float32),
        ],
        mesh=mesh,
        scratch_types=[
            pltpu.VMEM((nrow, 128), jnp.int32),
            pltpu.VMEM((nrow, 128), jnp.int32),
            pltpu.VMEM((nrow, 128), jnp.int32),
            pltpu.VMEM((_RING, _D, 128), jnp.float32),
            pltpu.VMEM((_RING, _D, 128), jnp.float32),
            pltpu.VMEM((_RING, _D, 128), jnp.float32),
            pltpu.VMEM((nrow, 128), jnp.float32),
            pltpu.VMEM((nrow, 128), jnp.float32),
            pltpu.SemaphoreType.DMA,
            pltpu.SemaphoreType.DMA,
            pltpu.SemaphoreType.DMA,
        ],
        compiler_params=pltpu.CompilerParams(needs_layout_passes=False),
    )
    def dots(iw_hbm, ow_hbm, nw_hbm, ei_hbm, eo_hbm, do_hbm, dn_hbm,
             iw_v, ow_v, nw_v, ib_v, ob_v, nb_v, do_v, dn_v,
             isem, osem, nsem):
        wid = lax.axis_index("s") * NC + lax.axis_index("c")
        pltpu.sync_copy(iw_hbm.at[wid], iw_v)
        pltpu.sync_copy(ow_hbm.at[wid], ow_v)
        pltpu.sync_copy(nw_hbm.at[wid], nw_v)

        def word_at(idx_v, j):
            v = plsc.load_gather(idx_v, [_splat(j // 128), _splat(j % 128)])
            return v[0]

        def fire(j, slot):
            wi = word_at(iw_v, j)
            wo = word_at(ow_v, j)
            wn = word_at(nw_v, j)
            for w, tab, buf, sem in (
                (wi, ei_hbm, ib_v, isem),
                (wo, eo_hbm, ob_v, osem),
                (wn, eo_hbm, nb_v, nsem),
            ):
                base = pl.multiple_of((w // 128) * 128, 128)
                pltpu.async_copy(tab.at[:, pl.ds(base, 128)], buf.at[slot], sem)

        def column(buf, slot, lane):
            lo = plsc.load_gather(
                buf, [_splat(slot), lax.iota(jnp.int32, _L), _splat(lane)])
            hi = plsc.load_gather(
                buf, [_splat(slot), _L + lax.iota(jnp.int32, _L), _splat(lane)])
            return lo, hi

        for j in range(_RING):
            fire(j, j)

        def body(j, carry):
            acc_o, acc_n = carry
            slot = j % _RING
            pltpu.make_async_copy(
                ei_hbm.at[:, pl.ds(0, 128)], ib_v.at[slot], isem).wait()
            pltpu.make_async_copy(
                eo_hbm.at[:, pl.ds(0, 128)], ob_v.at[slot], osem).wait()
            pltpu.make_async_copy(
                eo_hbm.at[:, pl.ds(0, 128)], nb_v.at[slot], nsem).wait()

            wi = word_at(iw_v, j)
            wo = word_at(ow_v, j)
            wn = word_at(nw_v, j)
            iv0, iv1 = column(ib_v, slot, wi % 128)
            ov0, ov1 = column(ob_v, slot, wo % 128)
            nv0, nv1 = column(nb_v, slot, wn % 128)

            @pl.when(j + _RING < bpw)
            def _():
                fire(j + _RING, slot)

            dot_o = jnp.sum(iv0 * ov0 + iv1 * ov1)
            dot_n = jnp.sum(iv0 * nv0 + iv1 * nv1)
            lane_mask = lax.iota(jnp.int32, _L) == _splat(j % _L)
            acc_o = jnp.where(lane_mask, jnp.full((_L,), dot_o), acc_o)
            acc_n = jnp.where(lane_mask, jnp.full((_L,), dot_n), acc_n)

            @pl.when(j % _L == _L - 1)
            def _():
                g = j // _L
                do_v[g // 8, pl.ds((g % 8) * _L, _L)] = acc_o
                dn_v[g // 8, pl.ds((g % 8) * _L, _L)] = acc_n

            return acc_o, acc_n

        zero = jnp.zeros((_L,), jnp.float32)
        lax.fori_loop(0, bpw, body, (zero, zero))
        pltpu.sync_copy(do_v, do_hbm.at[wid])
        pltpu.sync_copy(dn_v, dn_hbm.at[wid])

    return dots, NW, nrow


def _loss_body(do_ref, dn_ref, out_ref):
    x = do_ref[...]
    y = -dn_ref[...]
    ls = jnp.minimum(x, 0.0) - jnp.log1p(jnp.exp(-jnp.abs(x)))
    ls = ls + jnp.minimum(y, 0.0) - jnp.log1p(jnp.exp(-jnp.abs(y)))
    out_ref[0, 0] = -jnp.sum(ls) / _B


@functools.lru_cache(maxsize=1)
def _build_loss():
    return pl.pallas_call(
        _loss_body,
        out_shape=jax.ShapeDtypeStruct((1, 1), jnp.float32),
        out_specs=pl.BlockSpec(memory_space=pltpu.SMEM),
    )


@jax.jit
def kernel(iword, owords, nwords, emb_i, emb_o):
    dots, NW, nrow = _build_sc_dots()
    iw = iword.astype(jnp.int32).reshape(NW, nrow, 128)
    ow = owords.astype(jnp.int32).reshape(NW, nrow, 128)
    nw = nwords.astype(jnp.int32).reshape(NW, nrow, 128)
    do, dn = dots(iw, ow, nw, emb_i.T, emb_o.T)
    loss = _build_loss()(do.reshape(128, 128), dn.reshape(128, 128))
    return loss[0, 0]
